# baseline (speedup 1.0000x reference)
import functools

import jax
import jax.numpy as jnp
from jax import lax
from jax.experimental import pallas as pl
from jax.experimental.pallas import tpu as pltpu
from jax.experimental.pallas import tpu_sc as plsc

B = 4096
L = 200
N = B * L
WORD_DIM = 64
POS_DIM = 16
OUT_DIM = 96

NC = 2
NS = 16
NW = NC * NS
PER_W = N // NW
CHUNK = 512
NIT = PER_W // CHUNK


def _emb_body(word_hbm, pos1_hbm, pos2_hbm, wt_hbm, p1t_hbm, p2t_hbm,
              out_hbm, widx, p1idx, p2idx, p1t_v, p2t_v, ob,
              gsem, isem, wsem):
    c = lax.axis_index("c")
    s = lax.axis_index("s")
    wid = s * NC + c
    base0 = wid * PER_W

    pltpu.sync_copy(p1t_hbm, p1t_v)
    pltpu.sync_copy(p2t_hbm, p2t_v)

    pltpu.sync_copy(word_hbm.at[pl.ds(base0, CHUNK)], widx.at[pl.ds(0, CHUNK)])
    pltpu.sync_copy(pos1_hbm.at[pl.ds(base0, CHUNK)], p1idx.at[pl.ds(0, CHUNK)])
    pltpu.sync_copy(pos2_hbm.at[pl.ds(base0, CHUNK)], p2idx.at[pl.ds(0, CHUNK)])

    def body(it, carry):
        slot = lax.rem(it, 2)
        nslot = 1 - slot
        base = base0 + it * CHUNK
        soff = slot * CHUNK

        @pl.when(it + 1 < NIT)
        def _():
            nbase = base + CHUNK
            noff = nslot * CHUNK
            pltpu.async_copy(word_hbm.at[pl.ds(nbase, CHUNK)],
                             widx.at[pl.ds(noff, CHUNK)], isem)
            pltpu.async_copy(pos1_hbm.at[pl.ds(nbase, CHUNK)],
                             p1idx.at[pl.ds(noff, CHUNK)], isem)
            pltpu.async_copy(pos2_hbm.at[pl.ds(nbase, CHUNK)],
                             p2idx.at[pl.ds(noff, CHUNK)], isem)

        @pl.when(it >= 2)
        def _():
            pltpu.make_async_copy(
                ob.at[pl.ds(soff, CHUNK)],
                out_hbm.at[pl.ds(0, CHUNK)], wsem).wait()

        def group(g, carry2):
            r0 = soff + g * 16
            wv = widx[pl.ds(r0, 16)]
            p1v = p1idx[pl.ds(r0, 16)]
            p2v = p2idx[pl.ds(r0, 16)]
            for u in range(16):
                r = r0 + u
                pltpu.async_copy(
                    wt_hbm.at[wv[u]], ob.at[r, pl.ds(0, WORD_DIM)], gsem)
                ob[r, pl.ds(WORD_DIM, POS_DIM)] = p1t_v[p1v[u], :]
                ob[r, pl.ds(WORD_DIM + POS_DIM, POS_DIM)] = p2t_v[p2v[u], :]
            return carry2

        lax.fori_loop(0, CHUNK // 16, group, 0)

        pltpu.make_async_copy(
            wt_hbm.at[pl.ds(0, CHUNK)],
            ob.at[pl.ds(soff, CHUNK), pl.ds(0, WORD_DIM)], gsem).wait()

        @pl.when(it + 1 < NIT)
        def _():
            for ref in (widx, p1idx, p2idx):
                pltpu.make_async_copy(
                    word_hbm.at[pl.ds(0, CHUNK)],
                    ref.at[pl.ds(0, CHUNK)], isem).wait()

        pltpu.async_copy(ob.at[pl.ds(soff, CHUNK)],
                         out_hbm.at[pl.ds(base, CHUNK)], wsem)
        return carry

    lax.fori_loop(0, NIT, body, 0)

    for _ in range(2):
        pltpu.make_async_copy(
            ob.at[pl.ds(0, CHUNK)], out_hbm.at[pl.ds(0, CHUNK)], wsem).wait()


VOCAB = 1000000
TBLK = 128
NBLK = pl.cdiv(VOCAB, TBLK)


def _tx_body(wtT_ref, out_ref):
    blk = wtT_ref[...]
    pids = lax.broadcasted_iota(jnp.int32, (TBLK // 2, TBLK), 0)
    cols = lax.broadcasted_iota(jnp.int32, (TBLK // 2, TBLK), 1)
    sel_e = (cols == 2 * pids).astype(jnp.float32)
    sel_o = (cols == 2 * pids + 1).astype(jnp.float32)
    dn = (((1,), (1,)), ((), ()))
    left = lax.dot_general(sel_e, blk, dn,
                           precision=lax.Precision.HIGHEST,
                           preferred_element_type=jnp.float32)
    right = lax.dot_general(sel_o, blk, dn,
                            precision=lax.Precision.HIGHEST,
                            preferred_element_type=jnp.float32)
    out_ref[...] = jnp.concatenate([left, right], axis=1)


def _tx(wtT):
    return pl.pallas_call(
        _tx_body,
        grid=(NBLK,),
        in_specs=[pl.BlockSpec((WORD_DIM, TBLK), lambda i: (0, i))],
        out_specs=pl.BlockSpec((TBLK // 2, 128), lambda i: (i, 0)),
        out_shape=jax.ShapeDtypeStruct((VOCAB // 2, 128), jnp.float32),
    )(wtT)


@jax.jit
def _emb(word_f, pos1_f, pos2_f, word_table, pos1_table, pos2_table):
    mesh = plsc.VectorSubcoreMesh(core_axis_name="c", subcore_axis_name="s")
    f = pl.kernel(
        _emb_body,
        mesh=mesh,
        compiler_params=pltpu.CompilerParams(use_tc_tiling_on_sc=False),
        out_type=jax.ShapeDtypeStruct((N, OUT_DIM), jnp.float32),
        scratch_types=[
            pltpu.VMEM((2 * CHUNK,), jnp.int32),
            pltpu.VMEM((2 * CHUNK,), jnp.int32),
            pltpu.VMEM((2 * CHUNK,), jnp.int32),
            pltpu.VMEM((512, POS_DIM), jnp.float32),
            pltpu.VMEM((512, POS_DIM), jnp.float32),
            pltpu.VMEM((2 * CHUNK, OUT_DIM), jnp.float32),
            pltpu.SemaphoreType.DMA,
            pltpu.SemaphoreType.DMA,
            pltpu.SemaphoreType.DMA,
        ],
    )
    return f(word_f, pos1_f, pos2_f, word_table, pos1_table, pos2_table)


def kernel(word, pos1, pos2, word_table, pos1_table, pos2_table):
    word_f = word.astype(jnp.int32).reshape(N)
    pos1_f = pos1.astype(jnp.int32).reshape(N)
    pos2_f = pos2.astype(jnp.int32).reshape(N)
    wt_rm = _tx(word_table.T).reshape(VOCAB, WORD_DIM)
    out = _emb(word_f, pos1_f, pos2_f, wt_rm, pos1_table, pos2_table)
    return out.reshape(B, L, OUT_DIM)


# speedup vs baseline: 4.5930x; 4.5930x over previous
import functools

import jax
import jax.numpy as jnp
from jax import lax
from jax.experimental import pallas as pl
from jax.experimental.pallas import tpu as pltpu
from jax.experimental.pallas import tpu_sc as plsc

B = 4096
L = 200
N = B * L
WORD_DIM = 64
POS_DIM = 16
OUT_DIM = 96

NC = 2
NS = 16
NW = NC * NS
PER_W = N // NW
CHUNK = 512
NIT = PER_W // CHUNK


def _emb_body(word_hbm, pos1_hbm, pos2_hbm, wt_hbm, p1t_hbm, p2t_hbm,
              out_hbm, widx, p1idx, p2idx, p1t_v, p2t_v, ob,
              gsem, isem, wsem):
    c = lax.axis_index("c")
    s = lax.axis_index("s")
    wid = s * NC + c
    base0 = wid * PER_W

    pltpu.sync_copy(p1t_hbm, p1t_v)
    pltpu.sync_copy(p2t_hbm, p2t_v)

    pltpu.sync_copy(word_hbm.at[pl.ds(base0, CHUNK)], widx.at[pl.ds(0, CHUNK)])
    pltpu.sync_copy(pos1_hbm.at[pl.ds(base0, CHUNK)], p1idx.at[pl.ds(0, CHUNK)])
    pltpu.sync_copy(pos2_hbm.at[pl.ds(base0, CHUNK)], p2idx.at[pl.ds(0, CHUNK)])

    def body(it, carry):
        slot = lax.rem(it, 2)
        nslot = 1 - slot
        base = base0 + it * CHUNK
        soff = slot * CHUNK

        @pl.when(it + 1 < NIT)
        def _():
            nbase = base + CHUNK
            noff = nslot * CHUNK
            pltpu.async_copy(word_hbm.at[pl.ds(nbase, CHUNK)],
                             widx.at[pl.ds(noff, CHUNK)], isem)
            pltpu.async_copy(pos1_hbm.at[pl.ds(nbase, CHUNK)],
                             p1idx.at[pl.ds(noff, CHUNK)], isem)
            pltpu.async_copy(pos2_hbm.at[pl.ds(nbase, CHUNK)],
                             p2idx.at[pl.ds(noff, CHUNK)], isem)

        @pl.when(it >= 2)
        def _():
            pltpu.make_async_copy(
                ob.at[pl.ds(soff, CHUNK)],
                out_hbm.at[pl.ds(0, CHUNK)], wsem).wait()

        def group(g, carry2):
            r0 = soff + g * 16
            wv = widx[pl.ds(r0, 16)]
            p1v = p1idx[pl.ds(r0, 16)]
            p2v = p2idx[pl.ds(r0, 16)]
            for u in range(16):
                r = r0 + u
                v = wv[u]
                p = ((v >> 13) << 12) + (v & 4095)
                col = ((v >> 12) & 1) * WORD_DIM
                pltpu.async_copy(
                    wt_hbm.at[p, pl.ds(col, WORD_DIM)],
                    ob.at[r, pl.ds(0, WORD_DIM)], gsem)
                ob[r, pl.ds(WORD_DIM, POS_DIM)] = p1t_v[p1v[u], :]
                ob[r, pl.ds(WORD_DIM + POS_DIM, POS_DIM)] = p2t_v[p2v[u], :]
            return carry2

        lax.fori_loop(0, CHUNK // 16, group, 0)

        pltpu.make_async_copy(
            wt_hbm.at[pl.ds(0, CHUNK), pl.ds(0, WORD_DIM)],
            ob.at[pl.ds(soff, CHUNK), pl.ds(0, WORD_DIM)], gsem).wait()

        @pl.when(it + 1 < NIT)
        def _():
            for ref in (widx, p1idx, p2idx):
                pltpu.make_async_copy(
                    word_hbm.at[pl.ds(0, CHUNK)],
                    ref.at[pl.ds(0, CHUNK)], isem).wait()

        pltpu.async_copy(ob.at[pl.ds(soff, CHUNK)],
                         out_hbm.at[pl.ds(base, CHUNK)], wsem)
        return carry

    lax.fori_loop(0, NIT, body, 0)

    for _ in range(2):
        pltpu.make_async_copy(
            ob.at[pl.ds(0, CHUNK)], out_hbm.at[pl.ds(0, CHUNK)], wsem).wait()


VOCAB = 1000000
TBLK = 8192
NBLK = pl.cdiv(VOCAB, TBLK)
PACKED_ROWS = NBLK * (TBLK // 2)


def _tx_body(wtT_ref, out_ref):
    blk = wtT_ref[...]
    ta = lax.transpose(blk[:, : TBLK // 2], (1, 0))
    tb = lax.transpose(blk[:, TBLK // 2 :], (1, 0))
    out_ref[...] = jnp.concatenate([ta, tb], axis=1)


def _tx(wtT):
    return pl.pallas_call(
        _tx_body,
        grid=(NBLK,),
        in_specs=[pl.BlockSpec((WORD_DIM, TBLK), lambda i: (0, i))],
        out_specs=pl.BlockSpec((TBLK // 2, 128), lambda i: (i, 0)),
        out_shape=jax.ShapeDtypeStruct((PACKED_ROWS, 128), jnp.float32),
    )(wtT)


@jax.jit
def _emb(word_f, pos1_f, pos2_f, word_table, pos1_table, pos2_table):
    mesh = plsc.VectorSubcoreMesh(core_axis_name="c", subcore_axis_name="s")
    f = pl.kernel(
        _emb_body,
        mesh=mesh,
        compiler_params=pltpu.CompilerParams(use_tc_tiling_on_sc=False),
        out_type=jax.ShapeDtypeStruct((N, OUT_DIM), jnp.float32),
        scratch_types=[
            pltpu.VMEM((2 * CHUNK,), jnp.int32),
            pltpu.VMEM((2 * CHUNK,), jnp.int32),
            pltpu.VMEM((2 * CHUNK,), jnp.int32),
            pltpu.VMEM((512, POS_DIM), jnp.float32),
            pltpu.VMEM((512, POS_DIM), jnp.float32),
            pltpu.VMEM((2 * CHUNK, OUT_DIM), jnp.float32),
            pltpu.SemaphoreType.DMA,
            pltpu.SemaphoreType.DMA,
            pltpu.SemaphoreType.DMA,
        ],
    )
    return f(word_f, pos1_f, pos2_f, word_table, pos1_table, pos2_table)


def kernel(word, pos1, pos2, word_table, pos1_table, pos2_table):
    word_f = word.astype(jnp.int32).reshape(N)
    pos1_f = pos1.astype(jnp.int32).reshape(N)
    pos2_f = pos2.astype(jnp.int32).reshape(N)
    wt_rm = _tx(word_table.T)
    out = _emb(word_f, pos1_f, pos2_f, wt_rm, pos1_table, pos2_table)
    return out.reshape(B, L, OUT_DIM)


# speedup vs baseline: 6.3657x; 1.3859x over previous
import functools

import jax
import jax.numpy as jnp
from jax import lax
from jax.experimental import pallas as pl
from jax.experimental.pallas import tpu as pltpu
from jax.experimental.pallas import tpu_sc as plsc

B = 4096
L = 200
N = B * L
WORD_DIM = 64
POS_DIM = 16
OUT_DIM = 96
ROW = 128

NC = 2
NS = 16
NW = NC * NS
PER_W = N // NW
CHUNK = 256
NIT = PER_W // CHUNK
CPL = B // CHUNK

VOCAB = 1000000
TBLK = 8192
NBLK = pl.cdiv(VOCAB, TBLK)
PACKED_ROWS = NBLK * (TBLK // 2)


def _tx_body(wtT_ref, out_ref):
    blk = wtT_ref[...]
    ta = lax.transpose(blk[:, : TBLK // 2], (1, 0))
    tb = lax.transpose(blk[:, TBLK // 2 :], (1, 0))
    out_ref[...] = jnp.concatenate([ta, tb], axis=1)


def _tx(wtT):
    return pl.pallas_call(
        _tx_body,
        grid=(NBLK,),
        in_specs=[pl.BlockSpec((WORD_DIM, TBLK), lambda i: (0, i))],
        out_specs=pl.BlockSpec((TBLK // 2, 128), lambda i: (i, 0)),
        out_shape=jax.ShapeDtypeStruct((PACKED_ROWS, 128), jnp.float32),
    )(wtT)


def _ty_body(in_ref, out_ref):
    t = lax.transpose(in_ref[...], (1, 0))
    out_ref[...] = t[:OUT_DIM, :].reshape(1, OUT_DIM, B)


def _ty(packed):
    return pl.pallas_call(
        _ty_body,
        grid=(L,),
        in_specs=[pl.BlockSpec((B, ROW), lambda i: (i, 0))],
        out_specs=pl.BlockSpec((1, OUT_DIM, B), lambda i: (i, 0, 0)),
        out_shape=jax.ShapeDtypeStruct((L, OUT_DIM, B), jnp.float32),
    )(packed)


def _emb_body(word_hbm, pos1_hbm, pos2_hbm, wt_hbm, p1t_hbm, p2t_hbm,
              out_hbm, widx, p1idx, p2idx, p1t_v, p2t_v, ob,
              gsem, isem, wsem):
    c = lax.axis_index("c")
    s = lax.axis_index("s")
    wid = s * NC + c
    chunk0 = wid * NIT

    pltpu.sync_copy(p1t_hbm, p1t_v)
    pltpu.sync_copy(p2t_hbm, p2t_v)

    def idx_copy(cglob, slot, sem, copier):
        l = cglob // CPL
        b0 = (cglob % CPL) * CHUNK
        cps = []
        for src, dst in ((word_hbm, widx), (pos1_hbm, p1idx),
                         (pos2_hbm, p2idx)):
            cps.append(copier(src.at[l, pl.ds(b0, CHUNK)],
                              dst.at[pl.ds(slot * CHUNK, CHUNK)], sem))
        return cps

    for cp in idx_copy(chunk0, 0, isem, pltpu.async_copy):
        cp.wait()

    def body(it, carry):
        cglob = chunk0 + it
        slot = lax.rem(it, 2)
        nslot = 1 - slot
        soff = slot * CHUNK
        l = cglob // CPL
        b0 = (cglob % CPL) * CHUNK
        obase = l * B + b0

        @pl.when(it + 1 < NIT)
        def _():
            idx_copy(cglob + 1, nslot, isem, pltpu.async_copy)

        @pl.when(it >= 2)
        def _():
            pltpu.make_async_copy(
                ob.at[pl.ds(soff, CHUNK)],
                out_hbm.at[pl.ds(0, CHUNK)], wsem).wait()

        def group(g, carry2):
            r0 = soff + g * 16
            wv = widx[pl.ds(r0, 16)]
            p1v = p1idx[pl.ds(r0, 16)]
            p2v = p2idx[pl.ds(r0, 16)]
            for u in range(16):
                r = r0 + u
                v = wv[u]
                p = ((v >> 13) << 12) + (v & 4095)
                col = ((v >> 12) & 1) * WORD_DIM
                pltpu.async_copy(
                    wt_hbm.at[p, pl.ds(col, WORD_DIM)],
                    ob.at[r, pl.ds(0, WORD_DIM)], gsem)
                ob[r, pl.ds(WORD_DIM, POS_DIM)] = p1t_v[p1v[u], :]
                ob[r, pl.ds(WORD_DIM + POS_DIM, POS_DIM)] = p2t_v[p2v[u], :]
            return carry2

        lax.fori_loop(0, CHUNK // 16, group, 0)

        pltpu.make_async_copy(
            wt_hbm.at[pl.ds(0, CHUNK), pl.ds(0, WORD_DIM)],
            ob.at[pl.ds(soff, CHUNK), pl.ds(0, WORD_DIM)], gsem).wait()

        @pl.when(it + 1 < NIT)
        def _():
            for ref in (widx, p1idx, p2idx):
                pltpu.make_async_copy(
                    word_hbm.at[0, pl.ds(0, CHUNK)],
                    ref.at[pl.ds(0, CHUNK)], isem).wait()

        pltpu.async_copy(ob.at[pl.ds(soff, CHUNK)],
                         out_hbm.at[pl.ds(obase, CHUNK)], wsem)
        return carry

    lax.fori_loop(0, NIT, body, 0)

    for _ in range(2):
        pltpu.make_async_copy(
            ob.at[pl.ds(0, CHUNK)], out_hbm.at[pl.ds(0, CHUNK)], wsem).wait()


@jax.jit
def _emb(word_t, pos1_t, pos2_t, word_table, pos1_table, pos2_table):
    mesh = plsc.VectorSubcoreMesh(core_axis_name="c", subcore_axis_name="s")
    f = pl.kernel(
        _emb_body,
        mesh=mesh,
        compiler_params=pltpu.CompilerParams(use_tc_tiling_on_sc=False),
        out_type=jax.ShapeDtypeStruct((N, ROW), jnp.float32),
        scratch_types=[
            pltpu.VMEM((2 * CHUNK,), jnp.int32),
            pltpu.VMEM((2 * CHUNK,), jnp.int32),
            pltpu.VMEM((2 * CHUNK,), jnp.int32),
            pltpu.VMEM((512, POS_DIM), jnp.float32),
            pltpu.VMEM((512, POS_DIM), jnp.float32),
            pltpu.VMEM((2 * CHUNK, ROW), jnp.float32),
            pltpu.SemaphoreType.DMA,
            pltpu.SemaphoreType.DMA,
            pltpu.SemaphoreType.DMA,
        ],
    )
    return f(word_t, pos1_t, pos2_t, word_table, pos1_table, pos2_table)


def kernel(word, pos1, pos2, word_table, pos1_table, pos2_table):
    word_t = word.astype(jnp.int32).T
    pos1_t = pos1.astype(jnp.int32).T
    pos2_t = pos2.astype(jnp.int32).T
    wt_rm = _tx(word_table.T)
    packed = _emb(word_t, pos1_t, pos2_t, wt_rm, pos1_table, pos2_table)
    return _ty(packed).transpose(2, 0, 1)
